# baseline (speedup 1.0000x reference)
import jax
import jax.numpy as jnp
from jax.experimental import pallas as pl

N = 4096
BM = 512


def _leaky(x):
    return jnp.where(x >= 0, x, 0.25 * x)


def _proj_kernel(h_ref, w_ref, o_ref):
    o_ref[...] = jnp.dot(h_ref[...], w_ref[...],
                         preferred_element_type=jnp.float32).astype(jnp.bfloat16)


def _layer_fused_kernel(g_ref, s_ref, b_ref, w_ref, o_ref):
    t = jnp.dot(g_ref[...], s_ref[...], preferred_element_type=jnp.float32)
    t = _leaky(t + b_ref[...]).astype(jnp.bfloat16)
    o_ref[...] = jnp.dot(t, w_ref[...],
                         preferred_element_type=jnp.float32).astype(jnp.bfloat16)


def _layer_last_kernel(g_ref, s_ref, b_ref, o_ref):
    t = jnp.dot(g_ref[...], s_ref[...], preferred_element_type=jnp.float32)
    o_ref[...] = _leaky(t + b_ref[...]).astype(jnp.bfloat16)


def _decoder_kernel(hr_ref, tw_ref, hd_ref, o_ref):
    a = jnp.dot(hr_ref[...], tw_ref[...],
                preferred_element_type=jnp.float32).astype(jnp.bfloat16)
    o_ref[...] = jax.lax.dot_general(
        a, hd_ref[...], (((1,), (1,)), ((), ())),
        preferred_element_type=jnp.float32)


def _layer(G, S, b, W):
    hid = S.shape[1]
    b2d = b.reshape(1, hid)
    if W is None:
        return pl.pallas_call(
            _layer_last_kernel,
            grid=(N // BM,),
            in_specs=[
                pl.BlockSpec((BM, N), lambda i: (i, 0)),
                pl.BlockSpec((N, hid), lambda i: (0, 0)),
                pl.BlockSpec((1, hid), lambda i: (0, 0)),
            ],
            out_specs=pl.BlockSpec((BM, hid), lambda i: (i, 0)),
            out_shape=jax.ShapeDtypeStruct((N, hid), jnp.bfloat16),
        )(G, S, b2d)
    return pl.pallas_call(
        _layer_fused_kernel,
        grid=(N // BM,),
        in_specs=[
            pl.BlockSpec((BM, N), lambda i: (i, 0)),
            pl.BlockSpec((N, hid), lambda i: (0, 0)),
            pl.BlockSpec((1, hid), lambda i: (0, 0)),
            pl.BlockSpec((hid, hid), lambda i: (0, 0)),
        ],
        out_specs=pl.BlockSpec((BM, hid), lambda i: (i, 0)),
        out_shape=jax.ShapeDtypeStruct((N, hid), jnp.bfloat16),
    )(G, S, b2d, W)


def kernel(H, G, W1, b1, W2, b2, W3, b3, train_W, drug_num, target_num):
    n, in_dim = H.shape
    hid = W1.shape[1]

    Gb = G.astype(jnp.bfloat16)
    Hb = H.astype(jnp.bfloat16)
    W1b = W1.astype(jnp.bfloat16)
    W2b = W2.astype(jnp.bfloat16)
    W3b = W3.astype(jnp.bfloat16)
    tWb = train_W.astype(jnp.bfloat16)

    S1 = pl.pallas_call(
        _proj_kernel,
        grid=(n // BM,),
        in_specs=[
            pl.BlockSpec((BM, in_dim), lambda i: (i, 0)),
            pl.BlockSpec((in_dim, hid), lambda i: (0, 0)),
        ],
        out_specs=pl.BlockSpec((BM, hid), lambda i: (i, 0)),
        out_shape=jax.ShapeDtypeStruct((n, hid), jnp.bfloat16),
    )(Hb, W1b)

    S2 = _layer(Gb, S1, b1, W2b)
    S3 = _layer(Gb, S2, b2, W3b)
    h3 = _layer(Gb, S3, b3, None)

    d = n // 2
    t = n - d
    HR = jax.lax.dynamic_slice_in_dim(h3, drug_num - d, d)
    HD = jax.lax.dynamic_slice_in_dim(h3, drug_num + target_num - t, t)

    out = pl.pallas_call(
        _decoder_kernel,
        grid=(d // BM,),
        in_specs=[
            pl.BlockSpec((BM, hid), lambda i: (i, 0)),
            pl.BlockSpec((hid, hid), lambda i: (0, 0)),
            pl.BlockSpec((t, hid), lambda i: (0, 0)),
        ],
        out_specs=pl.BlockSpec((BM, t), lambda i: (i, 0)),
        out_shape=jax.ShapeDtypeStruct((d, t), jnp.float32),
    )(HR, tWb, HD)
    return out


# speedup vs baseline: 1.4643x; 1.4643x over previous
import jax
import jax.numpy as jnp
from jax.experimental import pallas as pl
from jax.experimental.pallas import tpu as pltpu

N = 4096
BM = 512
NB = N // BM


def _leaky(x):
    return jnp.where(x >= 0, x, 0.25 * x)


def _mega_kernel(g_ref, h_ref, w1_ref, b1_ref, w2_ref, b2_ref, w3_ref, b3_ref,
                 o_ref, gb_ref, sa_ref, sb_ref):
    s = pl.program_id(0)

    @pl.when(s < NB)
    def _phase0():
        gb_ref[pl.ds(s * BM, BM), :] = g_ref[...].astype(jnp.bfloat16)
        sa_ref[pl.ds(s * BM, BM), :] = jnp.dot(
            h_ref[...].astype(jnp.bfloat16), w1_ref[...],
            preferred_element_type=jnp.float32).astype(jnp.bfloat16)

    @pl.when((s >= NB) & (s < 2 * NB))
    def _phase1():
        i = s - NB
        t = jnp.dot(gb_ref[pl.ds(i * BM, BM), :], sa_ref[...],
                    preferred_element_type=jnp.float32)
        t = _leaky(t + b1_ref[...]).astype(jnp.bfloat16)
        sb_ref[pl.ds(i * BM, BM), :] = jnp.dot(
            t, w2_ref[...], preferred_element_type=jnp.float32
        ).astype(jnp.bfloat16)

    @pl.when((s >= 2 * NB) & (s < 3 * NB))
    def _phase2():
        i = s - 2 * NB
        t = jnp.dot(gb_ref[pl.ds(i * BM, BM), :], sb_ref[...],
                    preferred_element_type=jnp.float32)
        t = _leaky(t + b2_ref[...]).astype(jnp.bfloat16)
        sa_ref[pl.ds(i * BM, BM), :] = jnp.dot(
            t, w3_ref[...], preferred_element_type=jnp.float32
        ).astype(jnp.bfloat16)

    @pl.when(s >= 3 * NB)
    def _phase3():
        i = s - 3 * NB
        t = jnp.dot(gb_ref[pl.ds(i * BM, BM), :], sa_ref[...],
                    preferred_element_type=jnp.float32)
        o_ref[...] = _leaky(t + b3_ref[...]).astype(jnp.bfloat16)


def _decoder_kernel(hr_ref, tw_ref, hd_ref, o_ref):
    a = jnp.dot(hr_ref[...], tw_ref[...],
                preferred_element_type=jnp.float32).astype(jnp.bfloat16)
    o_ref[...] = jax.lax.dot_general(
        a, hd_ref[...], (((1,), (1,)), ((), ())),
        preferred_element_type=jnp.float32)


def kernel(H, G, W1, b1, W2, b2, W3, b3, train_W, drug_num, target_num):
    n, in_dim = H.shape
    hid = W1.shape[1]

    W1b = W1.astype(jnp.bfloat16)
    W2b = W2.astype(jnp.bfloat16)
    W3b = W3.astype(jnp.bfloat16)
    tWb = train_W.astype(jnp.bfloat16)
    b1r = b1.reshape(1, hid)
    b2r = b2.reshape(1, hid)
    b3r = b3.reshape(1, hid)

    h3 = pl.pallas_call(
        _mega_kernel,
        grid=(4 * NB,),
        in_specs=[
            pl.BlockSpec((BM, n), lambda s: (jnp.minimum(s, NB - 1), 0)),
            pl.BlockSpec((BM, in_dim), lambda s: (jnp.minimum(s, NB - 1), 0)),
            pl.BlockSpec((in_dim, hid), lambda s: (0, 0)),
            pl.BlockSpec((1, hid), lambda s: (0, 0)),
            pl.BlockSpec((hid, hid), lambda s: (0, 0)),
            pl.BlockSpec((1, hid), lambda s: (0, 0)),
            pl.BlockSpec((hid, hid), lambda s: (0, 0)),
            pl.BlockSpec((1, hid), lambda s: (0, 0)),
        ],
        out_specs=pl.BlockSpec(
            (BM, hid), lambda s: (jnp.maximum(s - 3 * NB, 0), 0)),
        out_shape=jax.ShapeDtypeStruct((n, hid), jnp.bfloat16),
        scratch_shapes=[
            pltpu.VMEM((n, n), jnp.bfloat16),
            pltpu.VMEM((n, hid), jnp.bfloat16),
            pltpu.VMEM((n, hid), jnp.bfloat16),
        ],
        compiler_params=pltpu.CompilerParams(
            vmem_limit_bytes=112 * 1024 * 1024),
    )(G, H, W1b, b1r, W2b, b2r, W3b, b3r)

    d = n // 2
    t = n - d
    HR = jax.lax.dynamic_slice_in_dim(h3, drug_num - d, d)
    HD = jax.lax.dynamic_slice_in_dim(h3, drug_num + target_num - t, t)

    out = pl.pallas_call(
        _decoder_kernel,
        grid=(d // BM,),
        in_specs=[
            pl.BlockSpec((BM, hid), lambda i: (i, 0)),
            pl.BlockSpec((hid, hid), lambda i: (0, 0)),
            pl.BlockSpec((t, hid), lambda i: (0, 0)),
        ],
        out_specs=pl.BlockSpec((BM, t), lambda i: (i, 0)),
        out_shape=jax.ShapeDtypeStruct((d, t), jnp.float32),
    )(HR, tWb, HD)
    return out


# speedup vs baseline: 1.5979x; 1.0912x over previous
import jax
import jax.numpy as jnp
from jax.experimental import pallas as pl
from jax.experimental.pallas import tpu as pltpu

N = 4096
BM = 512
NB = N // BM


def _leaky(x):
    return jnp.where(x >= 0, x, 0.25 * x)


def _mega_kernel(starts_ref, g_ref, h_ref, w1_ref, b1_ref, w2_ref, b2_ref,
                 w3_ref, b3_ref, tw_ref, o_ref, gb_ref, sa_ref, sb_ref):
    s = pl.program_id(0)

    @pl.when(s == 0)
    def _s1():
        sa_ref[...] = jnp.dot(
            h_ref[...], w1_ref[...],
            preferred_element_type=jnp.float32).astype(jnp.bfloat16)

    @pl.when((s >= 1) & (s < 1 + NB))
    def _stream_layer1():
        k = s - 1
        g = g_ref[...].astype(jnp.bfloat16)
        gb_ref[pl.ds(k * BM, BM), :] = g
        t = jnp.dot(g, sa_ref[...], preferred_element_type=jnp.float32)
        t = _leaky(t + b1_ref[...]).astype(jnp.bfloat16)
        sb_ref[pl.ds(k * BM, BM), :] = jnp.dot(
            t, w2_ref[...], preferred_element_type=jnp.float32
        ).astype(jnp.bfloat16)

    @pl.when((s >= 1 + NB) & (s < 1 + 2 * NB))
    def _layer2():
        i = s - (1 + NB)
        t = jnp.dot(gb_ref[pl.ds(i * BM, BM), :], sb_ref[...],
                    preferred_element_type=jnp.float32)
        t = _leaky(t + b2_ref[...]).astype(jnp.bfloat16)
        sa_ref[pl.ds(i * BM, BM), :] = jnp.dot(
            t, w3_ref[...], preferred_element_type=jnp.float32
        ).astype(jnp.bfloat16)

    @pl.when((s >= 1 + 2 * NB) & (s < 1 + 3 * NB))
    def _layer3():
        i = s - (1 + 2 * NB)
        t = jnp.dot(gb_ref[pl.ds(i * BM, BM), :], sa_ref[...],
                    preferred_element_type=jnp.float32)
        sb_ref[pl.ds(i * BM, BM), :] = _leaky(t + b3_ref[...]).astype(
            jnp.bfloat16)

    @pl.when(s >= 1 + 3 * NB)
    def _decoder():
        q = s - (1 + 3 * NB)
        j = q // 2
        c = q % 2
        hr0 = pl.multiple_of(starts_ref[0], BM)
        hd0 = pl.multiple_of(starts_ref[1], BM)
        hr = sb_ref[pl.ds(hr0 + j * BM, BM), :]
        a = jnp.dot(hr, tw_ref[...],
                    preferred_element_type=jnp.float32).astype(jnp.bfloat16)
        hd = sb_ref[pl.ds(hd0 + c * (N // 4), N // 4), :]
        o_ref[...] = jax.lax.dot_general(
            a, hd, (((1,), (1,)), ((), ())),
            preferred_element_type=jnp.float32)


def kernel(H, G, W1, b1, W2, b2, W3, b3, train_W, drug_num, target_num):
    n, in_dim = H.shape
    hid = W1.shape[1]
    d = n // 2
    t = n - d

    W1b = W1.astype(jnp.bfloat16)
    W2b = W2.astype(jnp.bfloat16)
    W3b = W3.astype(jnp.bfloat16)
    tWb = train_W.astype(jnp.bfloat16)
    b1r = b1.reshape(1, hid)
    b2r = b2.reshape(1, hid)
    b3r = b3.reshape(1, hid)
    starts = jnp.stack(
        [jnp.asarray(drug_num, jnp.int32) - d,
         jnp.asarray(drug_num, jnp.int32)
         + jnp.asarray(target_num, jnp.int32) - t])

    Hb = H.astype(jnp.bfloat16)
    dec0 = 1 + 3 * NB

    def _out_idx(s):
        q = jnp.maximum(s - dec0, 0)
        return (q // 2, q % 2)

    out = pl.pallas_call(
        _mega_kernel,
        grid=(dec0 + 2 * (d // BM),),
        in_specs=[
            pl.BlockSpec(memory_space=pltpu.SMEM),
            pl.BlockSpec((BM, n), lambda s: (jnp.clip(s - 1, 0, NB - 1), 0)),
            pl.BlockSpec((n, in_dim), lambda s: (0, 0)),
            pl.BlockSpec((in_dim, hid), lambda s: (0, 0)),
            pl.BlockSpec((1, hid), lambda s: (0, 0)),
            pl.BlockSpec((hid, hid), lambda s: (0, 0)),
            pl.BlockSpec((1, hid), lambda s: (0, 0)),
            pl.BlockSpec((hid, hid), lambda s: (0, 0)),
            pl.BlockSpec((1, hid), lambda s: (0, 0)),
            pl.BlockSpec((hid, hid), lambda s: (0, 0)),
        ],
        out_specs=pl.BlockSpec((BM, t // 2), _out_idx),
        out_shape=jax.ShapeDtypeStruct((d, t), jnp.float32),
        scratch_shapes=[
            pltpu.VMEM((n, n), jnp.bfloat16),
            pltpu.VMEM((n, hid), jnp.bfloat16),
            pltpu.VMEM((n, hid), jnp.bfloat16),
        ],
        compiler_params=pltpu.CompilerParams(
            vmem_limit_bytes=63 * 1024 * 1024),
    )(starts, G, Hb, W1b, b1r, W2b, b2r, W3b, b3r, tWb)
    return out


# speedup vs baseline: 1.6760x; 1.0489x over previous
import jax
import jax.numpy as jnp
from jax.experimental import pallas as pl
from jax.experimental.pallas import tpu as pltpu

N = 4096
BM = 512
NB = N // BM
BM2 = 1024
NB2 = N // BM2


def _leaky(x):
    return jnp.where(x >= 0, x, 0.25 * x)


def _mega_kernel(starts_ref, g_ref, h_ref, w1_ref, b1_ref, w2_ref, b2_ref,
                 w3_ref, b3_ref, tw_ref, o_ref, gb_ref, sa_ref, sb_ref):
    s = pl.program_id(0)

    @pl.when(s == 0)
    def _s1():
        sa_ref[...] = jnp.dot(
            h_ref[...], w1_ref[...],
            preferred_element_type=jnp.float32).astype(jnp.bfloat16)

    @pl.when((s >= 1) & (s < 1 + NB))
    def _stream_layer1():
        k = s - 1
        g = g_ref[...].astype(jnp.bfloat16)
        gb_ref[pl.ds(k * BM, BM), :] = g
        t = jnp.dot(g, sa_ref[...], preferred_element_type=jnp.float32)
        t = _leaky(t + b1_ref[...]).astype(jnp.bfloat16)
        sb_ref[pl.ds(k * BM, BM), :] = jnp.dot(
            t, w2_ref[...], preferred_element_type=jnp.float32
        ).astype(jnp.bfloat16)

    @pl.when((s >= 1 + NB) & (s < 1 + NB + NB2))
    def _layer2():
        i = s - (1 + NB)
        t = jnp.dot(gb_ref[pl.ds(i * BM2, BM2), :], sb_ref[...],
                    preferred_element_type=jnp.float32)
        t = _leaky(t + b2_ref[...]).astype(jnp.bfloat16)
        sa_ref[pl.ds(i * BM2, BM2), :] = jnp.dot(
            t, w3_ref[...], preferred_element_type=jnp.float32
        ).astype(jnp.bfloat16)

    @pl.when((s >= 1 + NB + NB2) & (s < 1 + NB + 2 * NB2))
    def _layer3():
        i = s - (1 + NB + NB2)
        t = jnp.dot(gb_ref[pl.ds(i * BM2, BM2), :], sa_ref[...],
                    preferred_element_type=jnp.float32)
        sb_ref[pl.ds(i * BM2, BM2), :] = _leaky(t + b3_ref[...]).astype(
            jnp.bfloat16)

    @pl.when(s >= 1 + NB + 2 * NB2)
    def _decoder():
        q = s - (1 + NB + 2 * NB2)
        j = q // 2
        c = q % 2
        hr0 = pl.multiple_of(starts_ref[0], BM)
        hd0 = pl.multiple_of(starts_ref[1], BM)
        hr = sb_ref[pl.ds(hr0 + j * BM, BM), :]
        a = jnp.dot(hr, tw_ref[...],
                    preferred_element_type=jnp.float32).astype(jnp.bfloat16)
        hd = sb_ref[pl.ds(hd0 + c * (N // 4), N // 4), :]
        o_ref[...] = jax.lax.dot_general(
            a, hd, (((1,), (1,)), ((), ())),
            preferred_element_type=jnp.float32)


def kernel(H, G, W1, b1, W2, b2, W3, b3, train_W, drug_num, target_num):
    n, in_dim = H.shape
    hid = W1.shape[1]
    d = n // 2
    t = n - d

    W1b = W1.astype(jnp.bfloat16)
    W2b = W2.astype(jnp.bfloat16)
    W3b = W3.astype(jnp.bfloat16)
    tWb = train_W.astype(jnp.bfloat16)
    b1r = b1.reshape(1, hid)
    b2r = b2.reshape(1, hid)
    b3r = b3.reshape(1, hid)
    starts = jnp.stack(
        [jnp.asarray(drug_num, jnp.int32) - d,
         jnp.asarray(drug_num, jnp.int32)
         + jnp.asarray(target_num, jnp.int32) - t])

    Hb = H.astype(jnp.bfloat16)
    dec0 = 1 + NB + 2 * NB2

    def _out_idx(s):
        q = jnp.maximum(s - dec0, 0)
        return (q // 2, q % 2)

    out = pl.pallas_call(
        _mega_kernel,
        grid=(dec0 + 2 * (d // BM),),
        in_specs=[
            pl.BlockSpec(memory_space=pltpu.SMEM),
            pl.BlockSpec((BM, n), lambda s: (jnp.clip(s - 1, 0, NB - 1), 0)),
            pl.BlockSpec((n, in_dim), lambda s: (0, 0)),
            pl.BlockSpec((in_dim, hid), lambda s: (0, 0)),
            pl.BlockSpec((1, hid), lambda s: (0, 0)),
            pl.BlockSpec((hid, hid), lambda s: (0, 0)),
            pl.BlockSpec((1, hid), lambda s: (0, 0)),
            pl.BlockSpec((hid, hid), lambda s: (0, 0)),
            pl.BlockSpec((1, hid), lambda s: (0, 0)),
            pl.BlockSpec((hid, hid), lambda s: (0, 0)),
        ],
        out_specs=pl.BlockSpec((BM, t // 2), _out_idx),
        out_shape=jax.ShapeDtypeStruct((d, t), jnp.float32),
        scratch_shapes=[
            pltpu.VMEM((n, n), jnp.bfloat16),
            pltpu.VMEM((n, hid), jnp.bfloat16),
            pltpu.VMEM((n, hid), jnp.bfloat16),
        ],
        compiler_params=pltpu.CompilerParams(
            vmem_limit_bytes=63 * 1024 * 1024),
    )(starts, G, Hb, W1b, b1r, W2b, b2r, W3b, b3r, tWb)
    return out
